# baseline (speedup 1.0000x reference)
import functools

import jax
import jax.numpy as jnp
from jax import lax
from jax.experimental import pallas as pl
from jax.experimental.pallas import tpu as pltpu
from jax.experimental.pallas import tpu_sc as plsc

_B = 4
_D = 2048

_mesh = plsc.VectorSubcoreMesh(core_axis_name="c", subcore_axis_name="s")
_NC = _mesh.num_cores


@functools.partial(
    pl.kernel,
    out_type=jax.ShapeDtypeStruct((_B, 2, _D), jnp.float32),
    mesh=_mesh,
    scratch_types=[pltpu.VMEM((_D,), jnp.float32)],
)
def _first_last_pool(h_hbm, out_hbm, row_buf):
    wid = lax.axis_index("s") * _NC + lax.axis_index("c")

    @pl.when(wid < _B)
    def _():
        pltpu.sync_copy(h_hbm.at[wid, 0, :], row_buf)
        pltpu.sync_copy(row_buf, out_hbm.at[wid, 0, :])
        pltpu.sync_copy(row_buf, out_hbm.at[wid, 1, :])


def kernel(hidden_state):
    return _first_last_pool(hidden_state)


# speedup vs baseline: 1.0705x; 1.0705x over previous
import functools

import jax
import jax.numpy as jnp
from jax import lax
from jax.experimental import pallas as pl
from jax.experimental.pallas import tpu as pltpu
from jax.experimental.pallas import tpu_sc as plsc

_B = 4
_D = 2048

_mesh = plsc.ScalarSubcoreMesh(axis_name="c", num_cores=1)


@functools.partial(
    pl.kernel,
    out_type=jax.ShapeDtypeStruct((_B, 2, _D), jnp.float32),
    mesh=_mesh,
    scratch_types=[pltpu.SemaphoreType.DMA],
)
def _first_last_pool(h_hbm, out_hbm, sem):
    copies = [
        pltpu.async_copy(h_hbm.at[b, 0, :], out_hbm.at[b, slot, :], sem)
        for b in range(_B)
        for slot in range(2)
    ]
    for c in copies:
        c.wait()


def kernel(hidden_state):
    return _first_last_pool(hidden_state)


# speedup vs baseline: 12.3325x; 11.5202x over previous
import jax
import jax.numpy as jnp
from jax.experimental import pallas as pl
from jax.experimental.pallas import tpu as pltpu

_B = 4
_D = 2048


def _body(h_ref, out_ref):
    row = h_ref[:, 0, :]
    out_ref[:, 0, :] = row
    out_ref[:, 1, :] = row


def kernel(hidden_state):
    return pl.pallas_call(
        _body,
        out_shape=jax.ShapeDtypeStruct((_B, 2, _D), jnp.float32),
        grid=(1,),
        in_specs=[
            pl.BlockSpec((_B, 8, _D), lambda i: (0, 0, 0)),
        ],
        out_specs=pl.BlockSpec((_B, 2, _D), lambda i: (0, 0, 0)),
    )(hidden_state)
